# baseline (speedup 1.0000x reference)
import functools

import jax
import jax.numpy as jnp
from jax import lax
from jax.experimental import pallas as pl
from jax.experimental.pallas import tpu as pltpu
from jax.experimental.pallas import tpu_sc as plsc


_NC = 2
_NS = 16
_ZR = 256
_NCP = 4


@functools.lru_cache(maxsize=None)
def _make_sc_op(rows, cols, dtype_name):
    dtype = jnp.dtype(dtype_name)
    nw = _NC * _NS
    rpw = rows // nw
    nwin = rpw // _ZR
    mesh = plsc.VectorSubcoreMesh(
        core_axis_name="c", subcore_axis_name="s",
        num_cores=_NC, num_subcores=_NS,
    )

    @functools.partial(
        pl.kernel,
        out_type=jax.ShapeDtypeStruct((2 * rows, cols), dtype),
        mesh=mesh,
        scratch_types=[
            pltpu.VMEM((_ZR, cols), dtype),
            pltpu.VMEM((_ZR, cols), dtype),
            pltpu.VMEM((_ZR, cols), dtype),
            pltpu.SemaphoreType.DMA,
            pltpu.SemaphoreType.DMA,
            pltpu.SemaphoreType.DMA,
            pltpu.SemaphoreType.DMA,
            pltpu.SemaphoreType.DMA,
        ],
    )
    def sc_op(x_hbm, o_hbm, bufa, bufb, zbuf, sia, sib, soa, sob, zsem):
        wid = lax.axis_index("s") * _NC + lax.axis_index("c")
        base = wid * rpw
        zero16 = jnp.zeros((16,), dtype)

        def zrow(i, carry):
            for j in range(cols // 16):
                zbuf[i, pl.ds(16 * j, 16)] = zero16
            return carry

        lax.fori_loop(0, _ZR, zrow, 0)

        zcopies = [
            pltpu.make_async_copy(
                zbuf,
                o_hbm.at[pl.ds(rows + base + k * _ZR, _ZR), :],
                zsem,
            )
            for k in range(nwin)
        ]
        for zc in zcopies:
            zc.start()

        del bufa, bufb, sia, sib, sob
        chunk = rpw // _NCP
        copies = [
            pltpu.make_async_copy(
                x_hbm.at[pl.ds(base + k * chunk, chunk), :],
                o_hbm.at[pl.ds(base + k * chunk, chunk), :],
                soa,
            )
            for k in range(_NCP)
        ]
        for c in copies:
            c.start()
        for c in copies:
            c.wait()

        for zc in zcopies:
            zc.wait()

    return sc_op


def kernel(psi):
    rows, cols = psi.shape
    return _make_sc_op(rows, cols, psi.dtype.name)(psi)


# speedup vs baseline: 12.4366x; 12.4366x over previous
import functools

import jax
import jax.numpy as jnp
from jax import lax
from jax.experimental import pallas as pl
from jax.experimental.pallas import tpu as pltpu
from jax.experimental.pallas import tpu_sc as plsc


_NC = 2
_NS = 16
_ZR = 256
_BLKN = 16384


@functools.lru_cache(maxsize=None)
def _make_sc_zero(rows, cols, dtype_name):
    dtype = jnp.dtype(dtype_name)
    nw = _NC * _NS
    rpw = rows // nw
    nz = rpw // _ZR
    mesh = plsc.VectorSubcoreMesh(
        core_axis_name="c", subcore_axis_name="s",
        num_cores=_NC, num_subcores=_NS,
    )

    @functools.partial(
        pl.kernel,
        out_type=jax.ShapeDtypeStruct((2 * rows, cols), dtype),
        mesh=mesh,
        scratch_types=[
            pltpu.VMEM((_ZR, cols), dtype),
            pltpu.SemaphoreType.DMA,
        ],
    )
    def sc_zero(o_hbm, zbuf, zsem):
        wid = lax.axis_index("s") * _NC + lax.axis_index("c")
        base = rows + wid * rpw
        zero16 = jnp.zeros((16,), dtype)

        def zrow(i, carry):
            for j in range(cols // 16):
                zbuf[i, pl.ds(16 * j, 16)] = zero16
            return carry

        lax.fori_loop(0, _ZR, zrow, 0)

        zcopies = [
            pltpu.make_async_copy(
                zbuf,
                o_hbm.at[pl.ds(base + k * _ZR, _ZR), :],
                zsem,
            )
            for k in range(nz)
        ]
        for zc in zcopies:
            zc.start()
        for zc in zcopies:
            zc.wait()

    return sc_zero


def _copy_body(x_ref, z_ref, o_ref):
    del z_ref
    o_ref[...] = x_ref[...]


def kernel(psi):
    rows, cols = psi.shape
    nb = rows // _BLKN

    zb = _make_sc_zero(rows, cols, psi.dtype.name)()

    return pl.pallas_call(
        _copy_body,
        grid=(nb,),
        in_specs=[
            pl.BlockSpec((_BLKN, cols), lambda i: (i, 0)),
            pl.BlockSpec(memory_space=pl.ANY),
        ],
        out_specs=pl.BlockSpec((_BLKN, cols), lambda i: (i, 0)),
        out_shape=jax.ShapeDtypeStruct((2 * rows, cols), psi.dtype),
        input_output_aliases={1: 0},
    )(psi, zb)


# speedup vs baseline: 12.4376x; 1.0001x over previous
import functools

import jax
import jax.numpy as jnp
from jax import lax
from jax.experimental import pallas as pl
from jax.experimental.pallas import tpu as pltpu
from jax.experimental.pallas import tpu_sc as plsc


_NC = 2
_NS = 16
_ZR = 512
_BLKN = 16384


@functools.lru_cache(maxsize=None)
def _make_sc_zero(rows, cols, dtype_name):
    dtype = jnp.dtype(dtype_name)
    nw = _NC * _NS
    rpw = rows // nw
    nz = rpw // _ZR
    mesh = plsc.VectorSubcoreMesh(
        core_axis_name="c", subcore_axis_name="s",
        num_cores=_NC, num_subcores=_NS,
    )

    @functools.partial(
        pl.kernel,
        out_type=jax.ShapeDtypeStruct((2 * rows, cols), dtype),
        mesh=mesh,
        scratch_types=[
            pltpu.VMEM((_ZR, cols), dtype),
            pltpu.SemaphoreType.DMA,
        ],
    )
    def sc_zero(o_hbm, zbuf, zsem):
        wid = lax.axis_index("s") * _NC + lax.axis_index("c")
        base = rows + wid * rpw
        zero16 = jnp.zeros((16,), dtype)

        def zrow(i, carry):
            for j in range(cols // 16):
                zbuf[i, pl.ds(16 * j, 16)] = zero16
            return carry

        lax.fori_loop(0, _ZR, zrow, 0)

        zcopies = [
            pltpu.make_async_copy(
                zbuf,
                o_hbm.at[pl.ds(base + k * _ZR, _ZR), :],
                zsem,
            )
            for k in range(nz)
        ]
        for zc in zcopies:
            zc.start()
        for zc in zcopies:
            zc.wait()

    return sc_zero


def _copy_body(x_ref, z_ref, o_ref):
    del z_ref
    o_ref[...] = x_ref[...]


def kernel(psi):
    rows, cols = psi.shape
    nb = rows // _BLKN

    zb = _make_sc_zero(rows, cols, psi.dtype.name)()

    return pl.pallas_call(
        _copy_body,
        grid=(nb,),
        in_specs=[
            pl.BlockSpec((_BLKN, cols), lambda i: (i, 0)),
            pl.BlockSpec(memory_space=pl.ANY),
        ],
        out_specs=pl.BlockSpec((_BLKN, cols), lambda i: (i, 0)),
        out_shape=jax.ShapeDtypeStruct((2 * rows, cols), psi.dtype),
        input_output_aliases={1: 0},
    )(psi, zb)
